# baseline (speedup 1.0000x reference)
import functools

import jax
import jax.numpy as jnp
from jax import lax
from jax.experimental import pallas as pl
from jax.experimental.pallas import tpu as pltpu
from jax.experimental.pallas import tpu_sc as plsc

NC, NS, L = 2, 16, 16


def _mm_body(x_ref, w_ref, o_ref):
  o_ref[...] = jnp.dot(x_ref[...], w_ref[...],
                       preferred_element_type=jnp.float32)


def _matmul(x, w, bn):
  n, f = x.shape
  k = w.shape[1]
  return pl.pallas_call(
      _mm_body,
      grid=(n // bn,),
      in_specs=[pl.BlockSpec((bn, f), lambda i: (i, 0)),
                pl.BlockSpec((f, k), lambda i: (0, 0))],
      out_specs=pl.BlockSpec((bn, k), lambda i: (i, 0)),
      out_shape=jax.ShapeDtypeStruct((n, k), jnp.float32),
  )(x, w)


def _elu(x):
  return jnp.where(x > 0, x, jnp.exp(jnp.minimum(x, 0.0)) - 1.0)


def _l1_combine_body(acc_ref, s_ref, w_ref, o_ref):
  a = acc_ref[0] + acc_ref[1]
  msg = a[:, 0:64]
  dn = a[:, 64:72]
  den = jnp.dot(dn, s_ref[...], preferred_element_type=jnp.float32) + 1e-9
  x1 = _elu(msg / den)
  o_ref[...] = jnp.dot(x1, w_ref[...], preferred_element_type=jnp.float32)


def _l2_combine_body(acc_a_ref, acc_b_ref, s2_ref, r_ref, o_ref):
  tot = None
  for acc_ref in (acc_a_ref, acc_b_ref):
    a = acc_ref[0] + acc_ref[1]
    msg = a[:, 0:128]
    dn = a[:, 128:132]
    den = jnp.dot(dn, s2_ref[...], preferred_element_type=jnp.float32) + 1e-9
    agg = msg / den
    tot = agg if tot is None else tot + agg
  x2 = jnp.dot(tot, r_ref[...], preferred_element_type=jnp.float32) * 0.125
  o_ref[...] = _elu(x2)


def _sc_mesh():
  return plsc.VectorSubcoreMesh(core_axis_name="c", subcore_axis_name="s",
                                num_cores=NC, num_subcores=NS)


def _make_edge_pass(n_nodes, n_edges, d, w, shifts, ch):
  epw = n_edges // (NC * NS)
  nchunks = epw // ch
  hh = ch // 2
  stripe = (n_nodes // NS) // 8 * 8
  tail = n_nodes - NS * stripe
  nregs = d // 16

  @functools.partial(
      pl.kernel,
      mesh=_sc_mesh(),
      compiler_params=pltpu.CompilerParams(use_tc_tiling_on_sc=False),
      out_type=jax.ShapeDtypeStruct((NC, n_nodes, w), jnp.float32),
      scratch_types=[
          pltpu.VMEM_SHARED((n_nodes, w), jnp.float32),
          pltpu.VMEM((nchunks + 1, 2 * ch), jnp.int32),
          pltpu.VMEM((nchunks, ch), jnp.int32),
          pltpu.VMEM((2 * hh, d), jnp.float32),
          pltpu.VMEM((2 * hh, d), jnp.float32),
          pltpu.VMEM((ch, w), jnp.float32),
          pltpu.VMEM((ch, w), jnp.float32),
          pltpu.VMEM((d,), jnp.float32),
          pltpu.VMEM((32,), jnp.float32),
          pltpu.SemaphoreType.DMA,
          pltpu.SemaphoreType.DMA,
          pltpu.SemaphoreType.DMA,
          pltpu.SemaphoreType.DMA,
      ],
  )
  def edge_pass(il_hbm, tgt_hbm, tab_hbm, av_hbm, zer_hbm, out_hbm,
                acc_sh, il_v, tgt_v, g_a, g_b, row_a, row_b, a_v, buf_v,
                sga, sgb, ssa, ssb):
    c = lax.axis_index("c")
    s = lax.axis_index("s")
    wid = s * NC + c
    pltpu.sync_copy(zer_hbm.at[pl.ds(0, stripe)],
                    acc_sh.at[pl.ds(s * stripe, stripe)])
    @pl.when(s == 0)
    def _():
      pltpu.sync_copy(zer_hbm.at[pl.ds(0, tail)],
                      acc_sh.at[pl.ds(NS * stripe, tail)])
    pltpu.sync_copy(av_hbm, a_v)
    pltpu.sync_copy(il_hbm.at[wid], il_v)
    pltpu.sync_copy(tgt_hbm.at[wid], tgt_v)
    plsc.subcore_barrier()

    a_regs = [a_v[pl.ds(j * 16, 16)] for j in range(nregs)]
    gs = ((g_a, sga), (g_b, sgb))
    rows = ((row_a, ssa), (row_b, ssb))

    def issue_g(ci, par):
      g_v, sem = gs[par]
      pltpu.async_copy(
          tab_hbm.at[il_v.at[ci, pl.ds(par * 2 * hh, 2 * hh)]],
          g_v, sem)

    def drain_g(par):
      g_v, sem = gs[par]
      pltpu.make_async_copy(tab_hbm.at[pl.ds(0, 2 * hh)], g_v, sem).wait()

    def drain_sc(k):
      row_v, sem = rows[k]
      pltpu.make_async_copy(zer_hbm.at[pl.ds(0, ch)], row_v, sem).wait()

    def compute_half(hbase, par, k):
      g_v, _ = gs[par]
      row_v, _ = rows[k]

      def edge(i, ecarry):
        t = None
        gs_regs = []
        for j in range(nregs):
          gt = g_v[2 * i, pl.ds(j * 16, 16)]
          gv = g_v[2 * i + 1, pl.ds(j * 16, 16)]
          gs_regs.append(gv)
          f = gt + gv
          f = jnp.maximum(f, 0.2 * f)
          p = f * a_regs[j]
          t = p if t is None else t + p
        for sh in shifts:
          buf_v[pl.ds(0, 16)] = t
          buf_v[pl.ds(16, 16)] = t
          t = t + buf_v[pl.ds(sh, 16)]
        ex = jnp.exp(t)
        for j in range(nregs):
          row_v[hbase + i, pl.ds(j * 16, 16)] = ex * gs_regs[j]
        row_v[hbase + i, pl.ds(d, 16)] = ex
        return ecarry

      lax.fori_loop(0, hh, edge, 0)

    def chunk_step(ci, k, with_drain):
      if with_drain:
        drain_sc(k)
      row_v, sem = rows[k]
      issue_g(ci, 1)
      drain_g(0)
      compute_half(0, 0, k)
      issue_g(ci + 1, 0)
      drain_g(1)
      compute_half(hh, 1, k)
      pltpu.async_copy(row_v, acc_sh.at[tgt_v.at[ci]], sem, add=True)

    issue_g(0, 0)
    chunk_step(0, 0, False)
    chunk_step(1, 1, False)

    def pair(pi, carry):
      chunk_step(2 * pi + 2, 0, True)
      chunk_step(2 * pi + 3, 1, True)
      return carry

    lax.fori_loop(0, (nchunks - 3) // 2, pair, 0)
    chunk_step(nchunks - 1, 0, True)
    drain_sc(1)
    drain_sc(0)
    drain_g(0)

    plsc.subcore_barrier()
    pltpu.sync_copy(acc_sh.at[pl.ds(s * stripe, stripe)],
                    out_hbm.at[c, pl.ds(s * stripe, stripe)])
    @pl.when(s == 0)
    def _():
      pltpu.sync_copy(acc_sh.at[pl.ds(NS * stripe, tail)],
                      out_hbm.at[c, pl.ds(NS * stripe, tail)])

  return edge_pass


def kernel(node_states, edges, indices, W1, a1, W2, a2):
  n, feat = node_states.shape
  e = edges.shape[0]
  h, _, u = W1.shape
  out = W2.shape[2]

  tgt = edges[:, 0]
  src = edges[:, 1]

  w1r = jnp.transpose(W1, (1, 2, 0)).reshape(feat, u * h)
  a1p = jnp.transpose(a1).reshape(u * h)
  w2r = jnp.transpose(W2.reshape(2, 4, h, u, out), (3, 2, 0, 4, 1))
  w2r = w2r.reshape(h * u, 2 * out * 4)
  a2p = jnp.transpose(a2.reshape(2, 4, out), (0, 2, 1)).reshape(2, 4 * out)

  s1 = (jnp.arange(64)[None, :] % 8 == jnp.arange(8)[:, None]).astype(
      jnp.float32)
  s2 = (jnp.arange(128)[None, :] % 4 == jnp.arange(4)[:, None]).astype(
      jnp.float32)
  rmat = (jnp.arange(128)[:, None] // 4 == jnp.arange(out)[None, :]).astype(
      jnp.float32)

  stripe = n // NS
  z1 = jnp.zeros((stripe, 80), jnp.float32)
  z2 = jnp.zeros((stripe, 144), jnp.float32)

  ch = 40
  nw = NC * NS
  nchunks = (e // nw) // ch
  il = jnp.concatenate(
      [edges.reshape(nw, nchunks, 2 * ch),
       jnp.zeros((nw, 1, 2 * ch), jnp.int32)], axis=1)
  tgtc = tgt.reshape(nw, nchunks, ch)

  g1 = _matmul(node_states, w1r, 1000)
  acc1 = _make_edge_pass(n, e, 64, 80, (8,), ch)(
      il, tgtc, g1, a1p, z1)
  g2t = pl.pallas_call(
      _l1_combine_body,
      grid=(10,),
      in_specs=[pl.BlockSpec((2, 1000, 80), lambda i: (0, i, 0)),
                pl.BlockSpec((8, 64), lambda i: (0, 0)),
                pl.BlockSpec((64, 256), lambda i: (0, 0))],
      out_specs=pl.BlockSpec((1000, 256), lambda i: (i, 0)),
      out_shape=jax.ShapeDtypeStruct((n, 256), jnp.float32),
  )(acc1, s1, w2r)

  l2_pass = _make_edge_pass(n, e, 128, 144, (8, 4), ch)
  acc2a = l2_pass(il, tgtc, g2t[:, 0:128], a2p[0], z2)
  acc2b = l2_pass(il, tgtc, g2t[:, 128:256], a2p[1], z2)

  x2 = pl.pallas_call(
      _l2_combine_body,
      grid=(10,),
      in_specs=[pl.BlockSpec((2, 1000, 144), lambda i: (0, i, 0)),
                pl.BlockSpec((2, 1000, 144), lambda i: (0, i, 0)),
                pl.BlockSpec((4, 128), lambda i: (0, 0)),
                pl.BlockSpec((128, 32), lambda i: (0, 0))],
      out_specs=pl.BlockSpec((1000, 32), lambda i: (i, 0)),
      out_shape=jax.ShapeDtypeStruct((n, out), jnp.float32),
  )(acc2a, acc2b, s2, rmat)

  return jnp.take(x2, indices, axis=0)


# speedup vs baseline: 1.0036x; 1.0036x over previous
import functools

import jax
import jax.numpy as jnp
from jax import lax
from jax.experimental import pallas as pl
from jax.experimental.pallas import tpu as pltpu
from jax.experimental.pallas import tpu_sc as plsc

NC, NS, L = 2, 16, 16


def _mm_body(x_ref, w_ref, o_ref):
  o_ref[...] = jnp.dot(x_ref[...], w_ref[...],
                       preferred_element_type=jnp.float32)


def _matmul(x, w, bn):
  n, f = x.shape
  k = w.shape[1]
  return pl.pallas_call(
      _mm_body,
      grid=(n // bn,),
      in_specs=[pl.BlockSpec((bn, f), lambda i: (i, 0)),
                pl.BlockSpec((f, k), lambda i: (0, 0))],
      out_specs=pl.BlockSpec((bn, k), lambda i: (i, 0)),
      out_shape=jax.ShapeDtypeStruct((n, k), jnp.float32),
  )(x, w)


def _elu(x):
  return jnp.where(x > 0, x, jnp.exp(jnp.minimum(x, 0.0)) - 1.0)


def _l1_combine_body(acc_ref, s_ref, w_ref, o_ref):
  a = acc_ref[0] + acc_ref[1]
  msg = a[:, 0:64]
  dn = a[:, 64:72]
  den = jnp.dot(dn, s_ref[...], preferred_element_type=jnp.float32) + 1e-9
  x1 = _elu(msg / den)
  o_ref[...] = jnp.dot(x1, w_ref[...], preferred_element_type=jnp.float32)


def _l2_combine_body(acc_a_ref, acc_b_ref, s2_ref, r_ref, o_ref):
  tot = None
  for acc_ref in (acc_a_ref, acc_b_ref):
    a = acc_ref[0] + acc_ref[1]
    msg = a[:, 0:128]
    dn = a[:, 128:132]
    den = jnp.dot(dn, s2_ref[...], preferred_element_type=jnp.float32) + 1e-9
    agg = msg / den
    tot = agg if tot is None else tot + agg
  x2 = jnp.dot(tot, r_ref[...], preferred_element_type=jnp.float32) * 0.125
  o_ref[...] = _elu(x2)


def _sc_mesh():
  return plsc.VectorSubcoreMesh(core_axis_name="c", subcore_axis_name="s",
                                num_cores=NC, num_subcores=NS)


def _make_edge_pass(n_nodes, n_edges, d, w, shifts, ch):
  epw = n_edges // (NC * NS)
  nchunks = epw // ch
  hh = ch // 2
  stripe = (n_nodes // NS) // 8 * 8
  tail = n_nodes - NS * stripe
  nregs = d // 16

  @functools.partial(
      pl.kernel,
      mesh=_sc_mesh(),
      compiler_params=pltpu.CompilerParams(use_tc_tiling_on_sc=False),
      out_type=jax.ShapeDtypeStruct((NC, n_nodes, w), jnp.float32),
      scratch_types=[
          pltpu.VMEM_SHARED((n_nodes, w), jnp.float32),
          pltpu.VMEM((nchunks + 1, 2 * ch), jnp.int32),
          pltpu.VMEM((nchunks, ch), jnp.int32),
          pltpu.VMEM((2 * hh, d), jnp.float32),
          pltpu.VMEM((2 * hh, d), jnp.float32),
          pltpu.VMEM((ch, w), jnp.float32),
          pltpu.VMEM((ch, w), jnp.float32),
          pltpu.VMEM((d,), jnp.float32),
          pltpu.VMEM((4 * 32,), jnp.float32),
          pltpu.SemaphoreType.DMA,
          pltpu.SemaphoreType.DMA,
          pltpu.SemaphoreType.DMA,
          pltpu.SemaphoreType.DMA,
      ],
  )
  def edge_pass(il_hbm, tgt_hbm, tab_hbm, av_hbm, zer_hbm, out_hbm,
                acc_sh, il_v, tgt_v, g_a, g_b, row_a, row_b, a_v, buf_v,
                sga, sgb, ssa, ssb):
    c = lax.axis_index("c")
    s = lax.axis_index("s")
    wid = s * NC + c
    pltpu.sync_copy(zer_hbm.at[pl.ds(0, stripe)],
                    acc_sh.at[pl.ds(s * stripe, stripe)])
    @pl.when(s == 0)
    def _():
      pltpu.sync_copy(zer_hbm.at[pl.ds(0, tail)],
                      acc_sh.at[pl.ds(NS * stripe, tail)])
    pltpu.sync_copy(av_hbm, a_v)
    pltpu.sync_copy(il_hbm.at[wid], il_v)
    pltpu.sync_copy(tgt_hbm.at[wid], tgt_v)
    plsc.subcore_barrier()

    a_regs = [a_v[pl.ds(j * 16, 16)] for j in range(nregs)]
    gs = ((g_a, sga), (g_b, sgb))
    rows = ((row_a, ssa), (row_b, ssb))

    def issue_g(ci, par):
      g_v, sem = gs[par]
      pltpu.async_copy(
          tab_hbm.at[il_v.at[ci, pl.ds(par * 2 * hh, 2 * hh)]],
          g_v, sem)

    def drain_g(par):
      g_v, sem = gs[par]
      pltpu.make_async_copy(tab_hbm.at[pl.ds(0, 2 * hh)], g_v, sem).wait()

    def drain_sc(k):
      row_v, sem = rows[k]
      pltpu.make_async_copy(zer_hbm.at[pl.ds(0, ch)], row_v, sem).wait()

    def compute_half(hbase, par, k):
      g_v, _ = gs[par]
      row_v, _ = rows[k]
      ur = 4

      def edge4(ii, ecarry):
        for uu in range(ur):
          i = ii * ur + uu
          bb = 32 * uu
          t = None
          gs_regs = []
          for j in range(nregs):
            gt = g_v[2 * i, pl.ds(j * 16, 16)]
            gv = g_v[2 * i + 1, pl.ds(j * 16, 16)]
            gs_regs.append(gv)
            f = gt + gv
            f = jnp.maximum(f, 0.2 * f)
            p = f * a_regs[j]
            t = p if t is None else t + p
          for sh in shifts:
            buf_v[pl.ds(bb, 16)] = t
            buf_v[pl.ds(bb + 16, 16)] = t
            t = t + buf_v[pl.ds(bb + sh, 16)]
          ex = jnp.exp(t)
          for j in range(nregs):
            row_v[hbase + i, pl.ds(j * 16, 16)] = ex * gs_regs[j]
          row_v[hbase + i, pl.ds(d, 16)] = ex
        return ecarry

      lax.fori_loop(0, hh // ur, edge4, 0)

    def chunk_step(ci, k, with_drain):
      if with_drain:
        drain_sc(k)
      row_v, sem = rows[k]
      issue_g(ci, 1)
      drain_g(0)
      compute_half(0, 0, k)
      issue_g(ci + 1, 0)
      drain_g(1)
      compute_half(hh, 1, k)
      pltpu.async_copy(row_v, acc_sh.at[tgt_v.at[ci]], sem, add=True)

    issue_g(0, 0)
    chunk_step(0, 0, False)
    chunk_step(1, 1, False)

    def pair(pi, carry):
      chunk_step(2 * pi + 2, 0, True)
      chunk_step(2 * pi + 3, 1, True)
      return carry

    lax.fori_loop(0, (nchunks - 3) // 2, pair, 0)
    chunk_step(nchunks - 1, 0, True)
    drain_sc(1)
    drain_sc(0)
    drain_g(0)

    plsc.subcore_barrier()
    pltpu.sync_copy(acc_sh.at[pl.ds(s * stripe, stripe)],
                    out_hbm.at[c, pl.ds(s * stripe, stripe)])
    @pl.when(s == 0)
    def _():
      pltpu.sync_copy(acc_sh.at[pl.ds(NS * stripe, tail)],
                      out_hbm.at[c, pl.ds(NS * stripe, tail)])

  return edge_pass


def kernel(node_states, edges, indices, W1, a1, W2, a2):
  n, feat = node_states.shape
  e = edges.shape[0]
  h, _, u = W1.shape
  out = W2.shape[2]

  tgt = edges[:, 0]
  src = edges[:, 1]

  w1r = jnp.transpose(W1, (1, 2, 0)).reshape(feat, u * h)
  a1p = jnp.transpose(a1).reshape(u * h)
  w2r = jnp.transpose(W2.reshape(2, 4, h, u, out), (3, 2, 0, 4, 1))
  w2r = w2r.reshape(h * u, 2 * out * 4)
  a2p = jnp.transpose(a2.reshape(2, 4, out), (0, 2, 1)).reshape(2, 4 * out)

  s1 = (jnp.arange(64)[None, :] % 8 == jnp.arange(8)[:, None]).astype(
      jnp.float32)
  s2 = (jnp.arange(128)[None, :] % 4 == jnp.arange(4)[:, None]).astype(
      jnp.float32)
  rmat = (jnp.arange(128)[:, None] // 4 == jnp.arange(out)[None, :]).astype(
      jnp.float32)

  stripe = n // NS
  z1 = jnp.zeros((stripe, 80), jnp.float32)
  z2 = jnp.zeros((stripe, 144), jnp.float32)

  ch = 40
  nw = NC * NS
  nchunks = (e // nw) // ch
  il = jnp.concatenate(
      [edges.reshape(nw, nchunks, 2 * ch),
       jnp.zeros((nw, 1, 2 * ch), jnp.int32)], axis=1)
  tgtc = tgt.reshape(nw, nchunks, ch)

  g1 = _matmul(node_states, w1r, 1000)
  acc1 = _make_edge_pass(n, e, 64, 80, (8,), ch)(
      il, tgtc, g1, a1p, z1)
  g2t = pl.pallas_call(
      _l1_combine_body,
      grid=(10,),
      in_specs=[pl.BlockSpec((2, 1000, 80), lambda i: (0, i, 0)),
                pl.BlockSpec((8, 64), lambda i: (0, 0)),
                pl.BlockSpec((64, 256), lambda i: (0, 0))],
      out_specs=pl.BlockSpec((1000, 256), lambda i: (i, 0)),
      out_shape=jax.ShapeDtypeStruct((n, 256), jnp.float32),
  )(acc1, s1, w2r)

  l2_pass = _make_edge_pass(n, e, 128, 144, (8, 4), ch)
  acc2a = l2_pass(il, tgtc, g2t[:, 0:128], a2p[0], z2)
  acc2b = l2_pass(il, tgtc, g2t[:, 128:256], a2p[1], z2)

  x2 = pl.pallas_call(
      _l2_combine_body,
      grid=(10,),
      in_specs=[pl.BlockSpec((2, 1000, 144), lambda i: (0, i, 0)),
                pl.BlockSpec((2, 1000, 144), lambda i: (0, i, 0)),
                pl.BlockSpec((4, 128), lambda i: (0, 0)),
                pl.BlockSpec((128, 32), lambda i: (0, 0))],
      out_specs=pl.BlockSpec((1000, 32), lambda i: (i, 0)),
      out_shape=jax.ShapeDtypeStruct((n, out), jnp.float32),
  )(acc2a, acc2b, s2, rmat)

  return jnp.take(x2, indices, axis=0)


# speedup vs baseline: 1.0439x; 1.0402x over previous
import functools

import jax
import jax.numpy as jnp
from jax import lax
from jax.experimental import pallas as pl
from jax.experimental.pallas import tpu as pltpu
from jax.experimental.pallas import tpu_sc as plsc

NC, NS, L = 2, 16, 16


def _mm_body(x_ref, w_ref, o_ref):
  o_ref[...] = jnp.dot(x_ref[...], w_ref[...],
                       preferred_element_type=jnp.float32)


def _matmul(x, w, bn):
  n, f = x.shape
  k = w.shape[1]
  return pl.pallas_call(
      _mm_body,
      grid=(n // bn,),
      in_specs=[pl.BlockSpec((bn, f), lambda i: (i, 0)),
                pl.BlockSpec((f, k), lambda i: (0, 0))],
      out_specs=pl.BlockSpec((bn, k), lambda i: (i, 0)),
      out_shape=jax.ShapeDtypeStruct((n, k), jnp.float32),
  )(x, w)


def _elu(x):
  return jnp.where(x > 0, x, jnp.exp(jnp.minimum(x, 0.0)) - 1.0)


def _l1_combine_body(acc_ref, s_ref, w_ref, o_ref):
  a = acc_ref[0] + acc_ref[1]
  msg = a[:, 0:64]
  dn = a[:, 64:72]
  den = jnp.dot(dn, s_ref[...], preferred_element_type=jnp.float32) + 1e-9
  x1 = _elu(msg / den)
  o_ref[...] = jnp.dot(x1, w_ref[...], preferred_element_type=jnp.float32)


def _l2_combine_body(acc_a_ref, acc_b_ref, s2_ref, r_ref, o_ref):
  tot = None
  for acc_ref in (acc_a_ref, acc_b_ref):
    a = acc_ref[0] + acc_ref[1]
    msg = a[:, 0:128]
    dn = a[:, 128:132]
    den = jnp.dot(dn, s2_ref[...], preferred_element_type=jnp.float32) + 1e-9
    agg = msg / den
    tot = agg if tot is None else tot + agg
  x2 = jnp.dot(tot, r_ref[...], preferred_element_type=jnp.float32) * 0.125
  o_ref[...] = _elu(x2)


def _sc_mesh():
  return plsc.VectorSubcoreMesh(core_axis_name="c", subcore_axis_name="s",
                                num_cores=NC, num_subcores=NS)


def _make_edge_pass(n_nodes, n_edges, d, w, shifts, ch):
  epw = n_edges // (NC * NS)
  nchunks = epw // ch
  npairs = (nchunks - 1) // 2
  stripe = (n_nodes // NS) // 8 * 8
  tail = n_nodes - NS * stripe
  nregs = d // 16

  @functools.partial(
      pl.kernel,
      mesh=_sc_mesh(),
      compiler_params=pltpu.CompilerParams(use_tc_tiling_on_sc=False),
      out_type=jax.ShapeDtypeStruct((NC, n_nodes, w), jnp.float32),
      scratch_types=[
          pltpu.VMEM_SHARED((n_nodes, w), jnp.float32),
          pltpu.VMEM((nchunks, 2 * ch), jnp.int32),
          pltpu.VMEM((2, ch), jnp.int32),
          pltpu.VMEM((2 * ch, d), jnp.float32),
          pltpu.VMEM((2 * ch, d), jnp.float32),
          pltpu.VMEM((ch, w), jnp.float32),
          pltpu.VMEM((d,), jnp.float32),
          pltpu.VMEM((32,), jnp.float32),
          pltpu.SemaphoreType.DMA,
          pltpu.SemaphoreType.DMA,
      ],
  )
  def edge_pass(il_hbm, tgt_hbm, tab_hbm, av_hbm, zer_hbm, out_hbm,
                acc_sh, il_v, tgt_v, g_a, g_b, row_v, a_v, buf_v,
                sem_a, sem_b):
    c = lax.axis_index("c")
    s = lax.axis_index("s")
    wid = s * NC + c
    pltpu.sync_copy(zer_hbm.at[pl.ds(0, stripe)],
                    acc_sh.at[pl.ds(s * stripe, stripe)])
    @pl.when(s == 0)
    def _():
      pltpu.sync_copy(zer_hbm.at[pl.ds(0, tail)],
                      acc_sh.at[pl.ds(NS * stripe, tail)])
    pltpu.sync_copy(av_hbm, a_v)
    pltpu.sync_copy(il_hbm.at[wid], il_v)
    plsc.subcore_barrier()

    a_regs = [a_v[pl.ds(j * 16, 16)] for j in range(nregs)]
    slots = ((g_a, sem_a, 0), (g_b, sem_b, 1))

    def issue(ci, slot):
      g_v, sem, k = slot
      pltpu.async_copy(tab_hbm.at[il_v.at[ci]], g_v, sem)
      pltpu.async_copy(tgt_hbm.at[wid, ci], tgt_v.at[k], sem)

    def drain(slot):
      g_v, sem, k = slot
      pltpu.make_async_copy(tab_hbm.at[pl.ds(0, 2 * ch)], g_v, sem).wait()
      pltpu.make_async_copy(tgt_hbm.at[0, 0], tgt_v.at[k], sem).wait()

    def compute(ci, slot):
      g_v, _, k = slot

      def edge(i, ecarry):
        t = None
        gs_regs = []
        for j in range(nregs):
          gt = g_v[2 * i, pl.ds(j * 16, 16)]
          gv = g_v[2 * i + 1, pl.ds(j * 16, 16)]
          gs_regs.append(gv)
          f = gt + gv
          f = jnp.maximum(f, 0.2 * f)
          p = f * a_regs[j]
          t = p if t is None else t + p
        for sh in shifts:
          buf_v[pl.ds(0, 16)] = t
          buf_v[pl.ds(16, 16)] = t
          t = t + buf_v[pl.ds(sh, 16)]
        ex = jnp.exp(t)
        for j in range(nregs):
          row_v[i, pl.ds(j * 16, 16)] = ex * gs_regs[j]
        row_v[i, pl.ds(d, 16)] = ex
        return ecarry

      lax.fori_loop(0, ch, edge, 0)
      pltpu.sync_copy(row_v, acc_sh.at[tgt_v.at[k]], add=True)

    issue(0, slots[0])

    def pair(p, carry):
      ci0 = 2 * p
      issue(ci0 + 1, slots[1])
      drain(slots[0])
      compute(ci0, slots[0])
      issue(ci0 + 2, slots[0])
      drain(slots[1])
      compute(ci0 + 1, slots[1])
      return carry

    lax.fori_loop(0, npairs, pair, 0)
    drain(slots[0])
    compute(nchunks - 1, slots[0])

    plsc.subcore_barrier()
    pltpu.sync_copy(acc_sh.at[pl.ds(s * stripe, stripe)],
                    out_hbm.at[c, pl.ds(s * stripe, stripe)])
    @pl.when(s == 0)
    def _():
      pltpu.sync_copy(acc_sh.at[pl.ds(NS * stripe, tail)],
                      out_hbm.at[c, pl.ds(NS * stripe, tail)])

  return edge_pass


def kernel(node_states, edges, indices, W1, a1, W2, a2):
  n, feat = node_states.shape
  e = edges.shape[0]
  h, _, u = W1.shape
  out = W2.shape[2]

  tgt = edges[:, 0]
  src = edges[:, 1]

  w1r = jnp.transpose(W1, (1, 2, 0)).reshape(feat, u * h)
  a1p = jnp.transpose(a1).reshape(u * h)
  w2r = jnp.transpose(W2.reshape(2, 4, h, u, out), (3, 2, 0, 4, 1))
  w2r = w2r.reshape(h * u, 2 * out * 4)
  a2p = jnp.transpose(a2.reshape(2, 4, out), (0, 2, 1)).reshape(2, 4 * out)

  s1 = (jnp.arange(64)[None, :] % 8 == jnp.arange(8)[:, None]).astype(
      jnp.float32)
  s2 = (jnp.arange(128)[None, :] % 4 == jnp.arange(4)[:, None]).astype(
      jnp.float32)
  rmat = (jnp.arange(128)[:, None] // 4 == jnp.arange(out)[None, :]).astype(
      jnp.float32)

  stripe = n // NS
  z1 = jnp.zeros((stripe, 80), jnp.float32)
  z2 = jnp.zeros((stripe, 144), jnp.float32)

  ch = 40
  nw = NC * NS
  il = edges.reshape(nw, (e // nw) // ch, 2 * ch)
  tgtc = tgt.reshape(nw, (e // nw) // ch, ch)

  g1 = _matmul(node_states, w1r, 1000)
  acc1 = _make_edge_pass(n, e, 64, 80, (8,), ch)(
      il, tgtc, g1, a1p, z1)
  g2t = pl.pallas_call(
      _l1_combine_body,
      grid=(10,),
      in_specs=[pl.BlockSpec((2, 1000, 80), lambda i: (0, i, 0)),
                pl.BlockSpec((8, 64), lambda i: (0, 0)),
                pl.BlockSpec((64, 256), lambda i: (0, 0))],
      out_specs=pl.BlockSpec((1000, 256), lambda i: (i, 0)),
      out_shape=jax.ShapeDtypeStruct((n, 256), jnp.float32),
  )(acc1, s1, w2r)

  l2_pass = _make_edge_pass(n, e, 128, 144, (8, 4), ch)
  acc2a = l2_pass(il, tgtc, g2t[:, 0:128], a2p[0], z2)
  acc2b = l2_pass(il, tgtc, g2t[:, 128:256], a2p[1], z2)

  x2 = pl.pallas_call(
      _l2_combine_body,
      grid=(10,),
      in_specs=[pl.BlockSpec((2, 1000, 144), lambda i: (0, i, 0)),
                pl.BlockSpec((2, 1000, 144), lambda i: (0, i, 0)),
                pl.BlockSpec((4, 128), lambda i: (0, 0)),
                pl.BlockSpec((128, 32), lambda i: (0, 0))],
      out_specs=pl.BlockSpec((1000, 32), lambda i: (i, 0)),
      out_shape=jax.ShapeDtypeStruct((n, out), jnp.float32),
  )(acc2a, acc2b, s2, rmat)

  return jnp.take(x2, indices, axis=0)
